# baseline (speedup 1.0000x reference)
import functools

import jax
import jax.numpy as jnp
from jax import lax
from jax.experimental import pallas as pl
from jax.experimental.pallas import tpu as pltpu
from jax.experimental.pallas import tpu_sc as plsc

E = 160000
NEI = 8
D_IN = 128
H = 128

NC = 2
NS = 16
L = 16
NW = NC * NS

CE = 16
CPW = 320
EPW = CE * CPW
E_PAD = NW * EPW
R = 2560
GRID = E_PAD // R

_f32 = jnp.float32
_i32 = jnp.int32


HS = 2048.0
US = 512.0


def _pack_rows(h, hu):
    hq = jnp.clip(h * HS, -32704.0, 32704.0).astype(_i32)
    uq = jnp.clip(hu * US, -32704.0, 32704.0).astype(_i32)
    return lax.shift_left(uq, 16) | (hq & 0xFFFF)


def _precompute_body(fmess_ref, wcat_ref, bcat_ref, ur_ref,
                     fzh_ref, fr_ref, hcat_ref):
    i = pl.program_id(0)
    fb = fmess_ref[...]
    F = jnp.dot(fb, wcat_ref[...], preferred_element_type=_f32) + bcat_ref[...]
    fz = F[:, :H]
    fr = F[:, H:2 * H]
    fh = F[:, 2 * H:]
    h1 = jax.nn.sigmoid(fz) * jnp.tanh(fh)
    rid = lax.broadcasted_iota(jnp.int32, (R, 1), 0) + i * R
    h1 = jnp.where(rid == 0, 0.0, h1)
    hu = jnp.dot(h1, ur_ref[...], preferred_element_type=_f32)
    fzh_ref[...] = jnp.concatenate([fz, fh], axis=1)
    fr_ref[...] = fr
    hcat_ref[...] = _pack_rows(h1, hu)


def _precompute_call(fmess_p, wcat_t, bcat, ur_t):
    return pl.pallas_call(
        _precompute_body,
        grid=(GRID,),
        in_specs=[
            pl.BlockSpec((R, D_IN), lambda i: (i, 0)),
            pl.BlockSpec((D_IN, 3 * H), lambda i: (0, 0)),
            pl.BlockSpec((1, 3 * H), lambda i: (0, 0)),
            pl.BlockSpec((H, H), lambda i: (0, 0)),
        ],
        out_specs=[
            pl.BlockSpec((R, 2 * H), lambda i: (i, 0)),
            pl.BlockSpec((R, H), lambda i: (i, 0)),
            pl.BlockSpec((R, H), lambda i: (i, 0)),
        ],
        out_shape=[
            jax.ShapeDtypeStruct((E_PAD, 2 * H), _f32),
            jax.ShapeDtypeStruct((E_PAD, H), _f32),
            jax.ShapeDtypeStruct((E_PAD, H), _i32),
        ],
        compiler_params=pltpu.CompilerParams(
            dimension_semantics=("parallel",)),
    )(fmess_p, wcat_t, bcat, ur_t)


def _combine_body(last, fzh_ref, sumcat_ref, wzh_ref, whh_ref, ur_ref,
                  out_ref):
    i = pl.program_id(0)
    sum_h = sumcat_ref[:, :H]
    sum_g = sumcat_ref[:, H:]
    z = jax.nn.sigmoid(
        fzh_ref[:, :H]
        + jnp.dot(sum_h, wzh_ref[...], preferred_element_type=_f32))
    pre = jnp.tanh(
        fzh_ref[:, H:]
        + jnp.dot(sum_g, whh_ref[...], preferred_element_type=_f32))
    h = (1.0 - z) * sum_h + z * pre
    rid = lax.broadcasted_iota(jnp.int32, (R, 1), 0) + i * R
    h = jnp.where(rid == 0, 0.0, h)
    if last:
        out_ref[...] = h
    else:
        hu = jnp.dot(h, ur_ref[...], preferred_element_type=_f32)
        out_ref[...] = _pack_rows(h, hu)


def _combine_call(last, fzh, sumcat, wzh_t, whh_t, ur_t):
    ow = H
    out_dt = _f32 if last else _i32
    return pl.pallas_call(
        functools.partial(_combine_body, last),
        grid=(GRID,),
        in_specs=[
            pl.BlockSpec((R, 2 * H), lambda i: (i, 0)),
            pl.BlockSpec((R, 2 * H), lambda i: (i, 0)),
            pl.BlockSpec((H, H), lambda i: (0, 0)),
            pl.BlockSpec((H, H), lambda i: (0, 0)),
            pl.BlockSpec((H, H), lambda i: (0, 0)),
        ],
        out_specs=pl.BlockSpec((R, ow), lambda i: (i, 0)),
        out_shape=jax.ShapeDtypeStruct((E_PAD, ow), out_dt),
        compiler_params=pltpu.CompilerParams(
            dimension_semantics=("parallel",)),
    )(fzh, sumcat, wzh_t, whh_t, ur_t)


def _sc_gather_body(hcat_hbm, bg2_hbm, nfr_hbm, out_hbm,
                    idx_all, rows_v, fr_v, o_v,
                    gsem0, gsem1, fsem0, fsem1, osem0, osem1):
    wid = lax.axis_index("s") * NC + lax.axis_index("c")
    ebase = wid * EPW
    gsems = (gsem0, gsem1)
    fsems = (fsem0, fsem1)
    osems = (osem0, osem1)

    pltpu.sync_copy(bg2_hbm.at[pl.ds(wid * CPW, CPW)], idx_all)
    pltpu.async_copy(hcat_hbm.at[idx_all.at[0]], rows_v.at[0], gsem0)
    pltpu.async_copy(nfr_hbm.at[pl.ds(ebase, CE)], fr_v.at[0], fsem0)

    def _phase(c, b):
        nb = 1 - b
        eb = ebase + c * CE

        pltpu.make_async_copy(hcat_hbm.at[idx_all.at[c]],
                              rows_v.at[b], gsems[b]).wait()
        pltpu.make_async_copy(nfr_hbm.at[pl.ds(eb, CE)],
                              fr_v.at[b], fsems[b]).wait()

        @pl.when(c + 1 < CPW)
        def _prefetch():
            pltpu.async_copy(hcat_hbm.at[idx_all.at[c + 1]],
                             rows_v.at[nb], gsems[nb])
            pltpu.async_copy(nfr_hbm.at[pl.ds(eb + CE, CE)],
                             fr_v.at[nb], fsems[nb])

        @pl.when(c >= 2)
        def _drain_out():
            pltpu.make_async_copy(o_v.at[b], out_hbm.at[pl.ds(ebase, CE)],
                                  osems[b]).wait()

        @pl.loop(0, CE)
        def _edge(e):
            for v in range(H // L):
                sl = pl.ds(v * L, L)
                slu = pl.ds(H + v * L, L)
                frv = fr_v[b, e, sl]
                ah = [jnp.zeros((L,), _f32), jnp.zeros((L,), _f32)]
                ag = [jnp.zeros((L,), _f32), jnp.zeros((L,), _f32)]
                for j in range(NEI):
                    w = rows_v[b, e * NEI + j, sl]
                    lo = lax.shift_right_arithmetic(lax.shift_left(w, 16), 16)
                    hi = lax.shift_right_arithmetic(w, 16)
                    hv = lo.astype(_f32)
                    uv = hi.astype(_f32) * (1.0 / US)
                    g = hv / (1.0 + jnp.exp(frv + uv))
                    ah[j % 2] = ah[j % 2] + hv
                    ag[j % 2] = ag[j % 2] + g
                o_v[b, e, sl] = (ah[0] + ah[1]) * (1.0 / HS)
                o_v[b, e, slu] = (ag[0] + ag[1]) * (1.0 / HS)

        pltpu.async_copy(o_v.at[b], out_hbm.at[pl.ds(eb, CE)], osems[b])

    @pl.loop(0, CPW, step=2)
    def _pair(c0):
        _phase(c0, 0)
        _phase(c0 + 1, 1)

    pltpu.make_async_copy(o_v.at[0], out_hbm.at[pl.ds(ebase, CE)],
                          osem0).wait()
    pltpu.make_async_copy(o_v.at[1], out_hbm.at[pl.ds(ebase, CE)],
                          osem1).wait()


def _sc_gather_call(hcat, bg2, nfr):
    mesh = plsc.VectorSubcoreMesh(
        core_axis_name="c", subcore_axis_name="s",
        num_cores=NC, num_subcores=NS)
    return pl.kernel(
        _sc_gather_body,
        out_type=jax.ShapeDtypeStruct((E_PAD, 2 * H), _f32),
        mesh=mesh,
        scratch_types=[
            pltpu.VMEM((CPW, CE * NEI), jnp.int32),
            pltpu.VMEM((2, CE * NEI, H), _i32),
            pltpu.VMEM((2, CE, H), _f32),
            pltpu.VMEM((2, CE, 2 * H), _f32),
            pltpu.SemaphoreType.DMA,
            pltpu.SemaphoreType.DMA,
            pltpu.SemaphoreType.DMA,
            pltpu.SemaphoreType.DMA,
            pltpu.SemaphoreType.DMA,
            pltpu.SemaphoreType.DMA,
        ],
    )(hcat, bg2, nfr)


def kernel(fmess, bgraph, W_z_w, W_z_b, W_r_w, U_r_w, U_r_b, W_h_w, W_h_b):
    wcat_t = jnp.concatenate(
        [W_z_w[:, :D_IN].T, -W_r_w.T, W_h_w[:, :D_IN].T], axis=1)
    bcat = jnp.concatenate([W_z_b, -U_r_b, W_h_b]).reshape(1, 3 * H)
    urn_t = -U_r_w.T
    wzh_t = W_z_w[:, D_IN:].T
    whh_t = W_h_w[:, D_IN:].T

    fmess_p = jnp.pad(fmess, ((0, E_PAD - E), (0, 0)))
    bg2 = jnp.pad(bgraph.reshape(-1),
                  (0, (E_PAD - E) * NEI)).reshape(-1, CE * NEI)

    fzh, nfr, hcat = _precompute_call(fmess_p, wcat_t, bcat, urn_t)

    sumcat = _sc_gather_call(hcat, bg2, nfr)
    hcat = _combine_call(False, fzh, sumcat, wzh_t, whh_t, urn_t)

    sumcat = _sc_gather_call(hcat, bg2, nfr)
    h = _combine_call(True, fzh, sumcat, wzh_t, whh_t, urn_t)

    return h[:E]


# speedup vs baseline: 1.0536x; 1.0536x over previous
import functools

import jax
import jax.numpy as jnp
from jax import lax
from jax.experimental import pallas as pl
from jax.experimental.pallas import tpu as pltpu
from jax.experimental.pallas import tpu_sc as plsc

E = 160000
NEI = 8
D_IN = 128
H = 128

NC = 2
NS = 16
L = 16
NW = NC * NS

CE = 16
CPW = 320
EPW = CE * CPW
E_PAD = NW * EPW
R = 2560
GRID = E_PAD // R

_f32 = jnp.float32


def _precompute_body(fmess_ref, wcat_ref, bcat_ref, ur_ref,
                     fzh_ref, fr_ref, hcat_ref):
    i = pl.program_id(0)
    fb = fmess_ref[...]
    F = jnp.dot(fb, wcat_ref[...], preferred_element_type=_f32) + bcat_ref[...]
    fz = F[:, :H]
    fr = F[:, H:2 * H]
    fh = F[:, 2 * H:]
    h1 = jax.nn.sigmoid(fz) * jnp.tanh(fh)
    rid = lax.broadcasted_iota(jnp.int32, (R, 1), 0) + i * R
    h1 = jnp.where(rid == 0, 0.0, h1)
    hu = jnp.dot(h1, ur_ref[...], preferred_element_type=_f32)
    fzh_ref[...] = jnp.concatenate([fz, fh], axis=1)
    fr_ref[...] = fr
    hcat_ref[...] = jnp.concatenate([h1, hu], axis=1)


def _precompute_call(fmess_p, wcat_t, bcat, ur_t):
    return pl.pallas_call(
        _precompute_body,
        grid=(GRID,),
        in_specs=[
            pl.BlockSpec((R, D_IN), lambda i: (i, 0)),
            pl.BlockSpec((D_IN, 3 * H), lambda i: (0, 0)),
            pl.BlockSpec((1, 3 * H), lambda i: (0, 0)),
            pl.BlockSpec((H, H), lambda i: (0, 0)),
        ],
        out_specs=[
            pl.BlockSpec((R, 2 * H), lambda i: (i, 0)),
            pl.BlockSpec((R, H), lambda i: (i, 0)),
            pl.BlockSpec((R, 2 * H), lambda i: (i, 0)),
        ],
        out_shape=[
            jax.ShapeDtypeStruct((E_PAD, 2 * H), _f32),
            jax.ShapeDtypeStruct((E_PAD, H), _f32),
            jax.ShapeDtypeStruct((E_PAD, 2 * H), _f32),
        ],
        compiler_params=pltpu.CompilerParams(
            dimension_semantics=("parallel",)),
    )(fmess_p, wcat_t, bcat, ur_t)


def _combine_body(last, fzh_ref, sumcat_ref, wzh_ref, whh_ref, ur_ref,
                  out_ref):
    i = pl.program_id(0)
    sum_h = sumcat_ref[:, :H]
    sum_g = sumcat_ref[:, H:]
    z = jax.nn.sigmoid(
        fzh_ref[:, :H]
        + jnp.dot(sum_h, wzh_ref[...], preferred_element_type=_f32))
    pre = jnp.tanh(
        fzh_ref[:, H:]
        + jnp.dot(sum_g, whh_ref[...], preferred_element_type=_f32))
    h = (1.0 - z) * sum_h + z * pre
    rid = lax.broadcasted_iota(jnp.int32, (R, 1), 0) + i * R
    h = jnp.where(rid == 0, 0.0, h)
    if last:
        out_ref[...] = h
    else:
        hu = jnp.dot(h, ur_ref[...], preferred_element_type=_f32)
        out_ref[...] = jnp.concatenate([h, hu], axis=1)


def _combine_call(last, fzh, sumcat, wzh_t, whh_t, ur_t):
    ow = H if last else 2 * H
    return pl.pallas_call(
        functools.partial(_combine_body, last),
        grid=(GRID,),
        in_specs=[
            pl.BlockSpec((R, 2 * H), lambda i: (i, 0)),
            pl.BlockSpec((R, 2 * H), lambda i: (i, 0)),
            pl.BlockSpec((H, H), lambda i: (0, 0)),
            pl.BlockSpec((H, H), lambda i: (0, 0)),
            pl.BlockSpec((H, H), lambda i: (0, 0)),
        ],
        out_specs=pl.BlockSpec((R, ow), lambda i: (i, 0)),
        out_shape=jax.ShapeDtypeStruct((E_PAD, ow), _f32),
        compiler_params=pltpu.CompilerParams(
            dimension_semantics=("parallel",)),
    )(fzh, sumcat, wzh_t, whh_t, ur_t)


def _sc_gather_body(hcat_hbm, bg2_hbm, nfr_hbm, out_hbm,
                    idx_all, rows_v, fr_v, o_v,
                    gsem0, gsem1, fsem0, fsem1, osem0, osem1):
    wid = lax.axis_index("s") * NC + lax.axis_index("c")
    ebase = wid * EPW
    gsems = (gsem0, gsem1)
    fsems = (fsem0, fsem1)
    osems = (osem0, osem1)

    pltpu.sync_copy(bg2_hbm.at[pl.ds(wid * CPW, CPW)], idx_all)
    pltpu.async_copy(hcat_hbm.at[idx_all.at[0]], rows_v.at[0], gsem0)
    pltpu.async_copy(nfr_hbm.at[pl.ds(ebase, CE)], fr_v.at[0], fsem0)

    def _phase(c, b):
        nb = 1 - b
        eb = ebase + c * CE

        pltpu.make_async_copy(hcat_hbm.at[idx_all.at[c]],
                              rows_v.at[b], gsems[b]).wait()
        pltpu.make_async_copy(nfr_hbm.at[pl.ds(eb, CE)],
                              fr_v.at[b], fsems[b]).wait()

        @pl.when(c + 1 < CPW)
        def _prefetch():
            pltpu.async_copy(hcat_hbm.at[idx_all.at[c + 1]],
                             rows_v.at[nb], gsems[nb])
            pltpu.async_copy(nfr_hbm.at[pl.ds(eb + CE, CE)],
                             fr_v.at[nb], fsems[nb])

        @pl.when(c >= 2)
        def _drain_out():
            pltpu.make_async_copy(o_v.at[b], out_hbm.at[pl.ds(ebase, CE)],
                                  osems[b]).wait()

        @pl.loop(0, CE)
        def _edge(e):
            for v in range(H // L):
                sl = pl.ds(v * L, L)
                slu = pl.ds(H + v * L, L)
                frv = fr_v[b, e, sl]
                acc_h = jnp.zeros((L,), _f32)
                acc_g = jnp.zeros((L,), _f32)
                for j in range(NEI):
                    hv = rows_v[b, e * NEI + j, sl]
                    uv = rows_v[b, e * NEI + j, slu]
                    g = hv / (1.0 + jnp.exp(frv + uv))
                    acc_h = acc_h + hv
                    acc_g = acc_g + g
                o_v[b, e, sl] = acc_h
                o_v[b, e, slu] = acc_g

        pltpu.async_copy(o_v.at[b], out_hbm.at[pl.ds(eb, CE)], osems[b])

    @pl.loop(0, CPW, step=2)
    def _pair(c0):
        _phase(c0, 0)
        _phase(c0 + 1, 1)

    pltpu.make_async_copy(o_v.at[0], out_hbm.at[pl.ds(ebase, CE)],
                          osem0).wait()
    pltpu.make_async_copy(o_v.at[1], out_hbm.at[pl.ds(ebase, CE)],
                          osem1).wait()


def _sc_gather_call(hcat, bg2, nfr):
    mesh = plsc.VectorSubcoreMesh(
        core_axis_name="c", subcore_axis_name="s",
        num_cores=NC, num_subcores=NS)
    return pl.kernel(
        _sc_gather_body,
        out_type=jax.ShapeDtypeStruct((E_PAD, 2 * H), _f32),
        mesh=mesh,
        scratch_types=[
            pltpu.VMEM((CPW, CE * NEI), jnp.int32),
            pltpu.VMEM((2, CE * NEI, 2 * H), _f32),
            pltpu.VMEM((2, CE, H), _f32),
            pltpu.VMEM((2, CE, 2 * H), _f32),
            pltpu.SemaphoreType.DMA,
            pltpu.SemaphoreType.DMA,
            pltpu.SemaphoreType.DMA,
            pltpu.SemaphoreType.DMA,
            pltpu.SemaphoreType.DMA,
            pltpu.SemaphoreType.DMA,
        ],
    )(hcat, bg2, nfr)


def kernel(fmess, bgraph, W_z_w, W_z_b, W_r_w, U_r_w, U_r_b, W_h_w, W_h_b):
    wcat_t = jnp.concatenate(
        [W_z_w[:, :D_IN].T, -W_r_w.T, W_h_w[:, :D_IN].T], axis=1)
    bcat = jnp.concatenate([W_z_b, -U_r_b, W_h_b]).reshape(1, 3 * H)
    urn_t = -U_r_w.T
    wzh_t = W_z_w[:, D_IN:].T
    whh_t = W_h_w[:, D_IN:].T

    fmess_p = jnp.pad(fmess, ((0, E_PAD - E), (0, 0)))
    bg2 = jnp.pad(bgraph.reshape(-1),
                  (0, (E_PAD - E) * NEI)).reshape(-1, CE * NEI)

    fzh, nfr, hcat = _precompute_call(fmess_p, wcat_t, bcat, urn_t)

    sumcat = _sc_gather_call(hcat, bg2, nfr)
    hcat = _combine_call(False, fzh, sumcat, wzh_t, whh_t, urn_t)

    sumcat = _sc_gather_call(hcat, bg2, nfr)
    h = _combine_call(True, fzh, sumcat, wzh_t, whh_t, urn_t)

    return h[:E]
